# Initial kernel scaffold; baseline (speedup 1.0000x reference)
import jax
import jax.numpy as jnp
from jax.experimental import pallas as pl


def kernel(logits, temperature, min_p, epsilon_cutoff, eta_cutoff):
    raise NotImplementedError("write your pallas kernel here")


# speedup vs baseline: 3.4102x; 3.4102x over previous
import functools

import jax
import jax.numpy as jnp
from jax.experimental import pallas as pl
from jax.experimental.pallas import tpu as pltpu

_TEMP_MIN = 2e-05
_NEG_INF = float("-inf")


def _body(t_ref, minp_ref, eps_ref, eta_ref, x_ref, out_ref, samp_ref):
    x = x_ref[...]
    R, V = x.shape
    t = jnp.maximum(t_ref[...], _TEMP_MIN)
    s = x / t
    m = jnp.max(s, axis=-1, keepdims=True)
    e = jnp.exp(s - m)
    z1 = jnp.sum(e, axis=-1, keepdims=True)

    idx = jax.lax.broadcasted_iota(jnp.int32, (R, V), 1)
    top_idx = jnp.min(jnp.where(s == m, idx, V), axis=-1, keepdims=True)
    is_top = idx == top_idx

    top_prob = 1.0 / z1
    mask1 = (e / z1) < (minp_ref[...] * top_prob)

    e2 = jnp.where(mask1, 0.0, e)
    z2 = jnp.sum(e2, axis=-1, keepdims=True)
    mask2 = mask1 | (((e2 / z2) < eps_ref[...]) & ~is_top)

    e3 = jnp.where(mask2, 0.0, e)
    z3 = jnp.sum(e3, axis=-1, keepdims=True)
    logz3 = jnp.log(z3)
    p3 = e3 / z3
    plogp = jnp.where(p3 > 0, p3 * (s - m - logz3), 0.0)
    neg_ent = jnp.sum(plogp, axis=-1, keepdims=True)
    eta = eta_ref[...]
    eps_eta = jnp.minimum(eta, jnp.sqrt(eta) * jnp.exp(neg_ent))
    mask3 = mask2 | ((p3 < eps_eta) & ~is_top)

    e4 = jnp.where(mask3, 0.0, e)
    z4 = jnp.sum(e4, axis=-1, keepdims=True)
    out_ref[...] = jnp.where(mask3, _NEG_INF, s - m - jnp.log(z4))
    samp_ref[...] = top_idx


def kernel(logits, temperature, min_p, epsilon_cutoff, eta_cutoff):
    B, V = logits.shape
    R = 8
    grid = (B // R,)
    row_spec = pl.BlockSpec((R, 1), lambda i: (i, 0))
    out = pl.pallas_call(
        _body,
        grid=grid,
        in_specs=[row_spec, row_spec, row_spec, row_spec,
                  pl.BlockSpec((R, V), lambda i: (i, 0))],
        out_specs=[pl.BlockSpec((R, V), lambda i: (i, 0)),
                   pl.BlockSpec((R, 1), lambda i: (i, 0))],
        out_shape=[jax.ShapeDtypeStruct((B, V), jnp.float32),
                   jax.ShapeDtypeStruct((B, 1), jnp.int32)],
    )(temperature.reshape(B, 1), min_p.reshape(B, 1),
      epsilon_cutoff.reshape(B, 1), eta_cutoff.reshape(B, 1), logits)
    return out[0], out[1].reshape(B)


# speedup vs baseline: 4.3857x; 1.2860x over previous
import jax
import jax.numpy as jnp
from jax.experimental import pallas as pl

_TEMP_MIN = 2e-05
_NEG_INF = float("-inf")


def _body(t_ref, minp_ref, eps_ref, eta_ref, x_ref, out_ref, samp_ref):
    x = x_ref[...]
    R, V = x.shape
    rt = 1.0 / jnp.maximum(t_ref[...], _TEMP_MIN)
    s = x * rt
    m = jnp.max(s, axis=-1, keepdims=True)

    idx = jax.lax.broadcasted_iota(jnp.int32, (R, V), 1)
    top_idx = jnp.min(jnp.where(s == m, idx, V), axis=-1, keepdims=True)

    sm = s - m
    e = jnp.exp(sm)

    lminp = jnp.log(minp_ref[...])
    z2 = jnp.sum(jnp.where(sm >= lminp, e, 0.0), axis=-1, keepdims=True)

    lthr2 = jnp.maximum(lminp, jnp.log(eps_ref[...] * z2))
    keep2 = sm >= lthr2
    z3 = jnp.sum(jnp.where(keep2, e, 0.0), axis=-1, keepdims=True)
    u3 = jnp.sum(jnp.where(keep2, e * sm, 0.0), axis=-1, keepdims=True)
    z3 = z3 + jnp.where(lthr2 <= 0.0, 0.0, 1.0)

    neg_ent = u3 / z3 - jnp.log(z3)
    eta = eta_ref[...]
    eps_eta = jnp.minimum(eta, jnp.sqrt(eta) * jnp.exp(neg_ent))
    lthr3 = jnp.maximum(lthr2, jnp.log(eps_eta * z3))
    keep3 = sm >= lthr3
    z4 = jnp.sum(jnp.where(keep3, e, 0.0), axis=-1, keepdims=True)
    z4 = z4 + jnp.where(lthr3 <= 0.0, 0.0, 1.0)

    out_ref[...] = jnp.where(keep3 | (idx == top_idx),
                             sm - jnp.log(z4), _NEG_INF)
    samp_ref[...] = top_idx


def kernel(logits, temperature, min_p, epsilon_cutoff, eta_cutoff):
    B, V = logits.shape
    R = 8
    grid = (B // R,)
    row_spec = pl.BlockSpec((R, 1), lambda i: (i, 0))
    out = pl.pallas_call(
        _body,
        grid=grid,
        in_specs=[row_spec, row_spec, row_spec, row_spec,
                  pl.BlockSpec((R, V), lambda i: (i, 0))],
        out_specs=[pl.BlockSpec((R, V), lambda i: (i, 0)),
                   pl.BlockSpec((R, 1), lambda i: (i, 0))],
        out_shape=[jax.ShapeDtypeStruct((B, V), jnp.float32),
                   jax.ShapeDtypeStruct((B, 1), jnp.int32)],
    )(temperature.reshape(B, 1), min_p.reshape(B, 1),
      epsilon_cutoff.reshape(B, 1), eta_cutoff.reshape(B, 1), logits)
    return out[0], out[1].reshape(B)


# speedup vs baseline: 4.3906x; 1.0011x over previous
import jax
import jax.numpy as jnp
from jax.experimental import pallas as pl

_TEMP_MIN = 2e-05
_NEG_INF = float("-inf")


def _body(t_ref, minp_ref, eps_ref, eta_ref, x_ref, out_ref, samp_ref):
    x = x_ref[...]
    R, V = x.shape
    rt = 1.0 / jnp.maximum(t_ref[...], _TEMP_MIN)
    s = x * rt
    m = jnp.max(s, axis=-1, keepdims=True)

    sm = s - m
    e = jnp.exp(sm)

    idx = jax.lax.broadcasted_iota(jnp.int32, (R, V), 1)
    top_idx = jnp.min(jnp.where(sm == 0.0, idx, V), axis=-1, keepdims=True)

    lminp = jnp.log(minp_ref[...])
    z2 = jnp.sum(jnp.where(sm >= lminp, e, 0.0), axis=-1, keepdims=True)

    lthr2 = jnp.maximum(lminp, jnp.log(eps_ref[...] * z2))
    keep2 = sm >= lthr2
    z3 = jnp.sum(jnp.where(keep2, e, 0.0), axis=-1, keepdims=True)
    u3 = jnp.sum(jnp.where(keep2, e * sm, 0.0), axis=-1, keepdims=True)
    z3 = z3 + jnp.where(lthr2 <= 0.0, 0.0, 1.0)

    neg_ent = u3 / z3 - jnp.log(z3)
    eta = eta_ref[...]
    eps_eta = jnp.minimum(eta, jnp.sqrt(eta) * jnp.exp(neg_ent))
    lthr3 = jnp.maximum(lthr2, jnp.log(eps_eta * z3))
    keep3 = sm >= lthr3
    z4 = jnp.sum(jnp.where(keep3, e, 0.0), axis=-1, keepdims=True)
    z4 = z4 + jnp.where(lthr3 <= 0.0, 0.0, 1.0)

    out_ref[...] = jnp.where(keep3 | (idx == top_idx),
                             sm - jnp.log(z4), _NEG_INF)
    samp_ref[...] = top_idx


def kernel(logits, temperature, min_p, epsilon_cutoff, eta_cutoff):
    B, V = logits.shape
    R = 8
    grid = (B // R,)
    row_spec = pl.BlockSpec((R, 1), lambda i: (i, 0))
    out = pl.pallas_call(
        _body,
        grid=grid,
        in_specs=[row_spec, row_spec, row_spec, row_spec,
                  pl.BlockSpec((R, V), lambda i: (i, 0))],
        out_specs=[pl.BlockSpec((R, V), lambda i: (i, 0)),
                   pl.BlockSpec((R, 1), lambda i: (i, 0))],
        out_shape=[jax.ShapeDtypeStruct((B, V), jnp.float32),
                   jax.ShapeDtypeStruct((B, 1), jnp.int32)],
    )(temperature.reshape(B, 1), min_p.reshape(B, 1),
      epsilon_cutoff.reshape(B, 1), eta_cutoff.reshape(B, 1), logits)
    return out[0], out[1].reshape(B)
